# baseline (speedup 1.0000x reference)
import functools

import jax
import jax.numpy as jnp
from jax import lax
from jax.experimental import pallas as pl
from jax.experimental.pallas import tpu as pltpu
from jax.experimental.pallas import tpu_sc as plsc

_NC = 2
_NS = 16
_NW = _NC * _NS
_NBUF = 4
_FD = 2


@functools.cache
def _build_gather(b_total: int, l_total: int, dim: int):
    rows_per_w = b_total // _NW
    c0 = min(l_total, 128)
    c1 = l_total - c0
    n_chunks = 2 * rows_per_w
    n_super = n_chunks // _NBUF
    sz = {0: c0, 1: c1}
    off = {0: 0, 1: c0}
    mesh = plsc.VectorSubcoreMesh(
        core_axis_name="c", subcore_axis_name="s",
        num_cores=_NC, num_subcores=_NS,
    )

    @functools.partial(
        pl.kernel,
        out_type=jax.ShapeDtypeStruct((b_total, l_total, dim), jnp.float32),
        mesh=mesh,
        scratch_types=[
            pltpu.VMEM((rows_per_w, l_total), jnp.int32),
            pltpu.VMEM((_NBUF, c0, dim), jnp.float32),
        ]
        + [pltpu.SemaphoreType.DMA] * (2 * _NBUF),
        compiler_params=pltpu.CompilerParams(use_tc_tiling_on_sc=False),
    )
    def gather(idx_hbm, table_hbm, out_hbm, idx_v, rows_v, *sems):
        gs = sems[:_NBUF]
        os_ = sems[_NBUF:]
        wid = lax.axis_index("s") * _NC + lax.axis_index("c")
        base = wid * rows_per_w

        pltpu.sync_copy(idx_hbm.at[pl.ds(base, rows_per_w)], idx_v)

        def fire(c, b):
            p = b % 2
            pltpu.async_copy(
                table_hbm.at[idx_v.at[c // 2, pl.ds(off[p], sz[p])]],
                rows_v.at[b, pl.ds(0, sz[p])], gs[b])

        def drain_gather(b):
            p = b % 2
            pltpu.make_async_copy(
                table_hbm.at[idx_v.at[0, pl.ds(off[p], sz[p])]],
                rows_v.at[b, pl.ds(0, sz[p])], gs[b]).wait()

        def out_start(c, b):
            p = b % 2
            pltpu.async_copy(
                rows_v.at[b, pl.ds(0, sz[p])],
                out_hbm.at[base + c // 2, pl.ds(off[p], sz[p])], os_[b])

        def out_wait(b):
            p = b % 2
            pltpu.make_async_copy(
                rows_v.at[b, pl.ds(0, sz[p])],
                out_hbm.at[0, pl.ds(off[p], sz[p])], os_[b]).wait()

        for cp in range(_FD):
            fire(cp, cp)

        def super_iter(s, carry):
            for b in range(_NBUF):
                g = s * _NBUF + b
                bw = (b + _FD) % _NBUF
                if b + _FD < _NBUF:
                    @pl.when(s >= 1)
                    def _():
                        out_wait(bw)
                        fire(g + _FD, bw)
                    @pl.when(s == 0)
                    def _():
                        fire(g + _FD, bw)
                else:
                    out_wait(bw)
                    @pl.when(s < n_super - 1)
                    def _():
                        fire(g + _FD, bw)
                drain_gather(b)
                out_start(g, b)
            return carry

        lax.fori_loop(0, n_super, super_iter, 0)

        for j in range(_NBUF - _FD):
            out_wait((_FD + j) % _NBUF)

    return gather


def kernel(item_seq, item_emb_weight):
    b, l = item_seq.shape
    dim = item_emb_weight.shape[1]
    idx = item_seq.astype(jnp.int32)
    return _build_gather(b, l, dim)(idx, item_emb_weight)
